# baseline (speedup 1.0000x reference)
import jax
import jax.numpy as jnp
from jax.experimental import pallas as pl
from jax.experimental.pallas import tpu as pltpu
from functools import partial

_SIGMA = 2.0
_EPS = 1e-8
_INV2S2 = 1.0 / (2.0 * _SIGMA * _SIGMA)
_NLOG2E = -1.4426950408889634


def _body(pts_ref, logits_ref, out_ref, m_acc, *, n, p, zb, n_zb, y, x):
    i = pl.program_id(0)
    j = pl.program_id(1)

    pts = pts_ref[i]
    pz = pts[:, 0].reshape(1, p)
    py = pts[:, 1].reshape(1, p)
    px = pts[:, 2].reshape(1, p)

    zvals = (
        jax.lax.broadcasted_iota(jnp.int32, (zb, 1), 0) + j * zb
    ).astype(jnp.float32)
    kz = jnp.exp(-((zvals - pz) ** 2) * _INV2S2)
    yvals = jax.lax.broadcasted_iota(jnp.int32, (y, 1), 0).astype(jnp.float32)
    ky = jnp.exp(-((yvals - py) ** 2) * _INV2S2)
    xvals = jax.lax.broadcasted_iota(jnp.int32, (x, 1), 0).astype(jnp.float32)
    kx = jnp.exp(-((xvals - px) ** 2) * _INV2S2)

    a = (kz[:, None, :] * ky[None, :, :]).reshape(zb * y, p)

    denom = jax.lax.dot_general(
        a, kx, (((1,), (1,)), ((), ())),
        preferred_element_type=jnp.float32,
    )

    l = logits_ref[0].reshape(zb * y, x)
    e = jnp.exp2(l * _NLOG2E)
    w = 1.0 / ((1.0 + e) * jnp.maximum(denom, _EPS))

    t = jax.lax.dot_general(
        w, kx, (((1,), (0,)), ((), ())),
        preferred_element_type=jnp.float32,
    )
    contrib = jnp.sum(a * t, axis=0, keepdims=True)

    @pl.when(j == 0)
    def _():
        m_acc[...] = jnp.zeros_like(m_acc)

    m_acc[...] += contrib

    @pl.when(jnp.logical_and(i == 0, j == 0))
    def _():
        out_ref[...] = jnp.zeros_like(out_ref)

    @pl.when(j == n_zb - 1)
    def _():
        pt = -jnp.log(jnp.maximum(m_acc[...], _EPS))
        out_ref[...] += jnp.sum(pt) * (1.0 / (p * n))


@jax.jit
def kernel(logits, pts):
    n, _, z, y, x = logits.shape
    p = pts.shape[1]
    n_zb = 2
    zb = z // n_zb
    logits4 = logits.reshape(n, z, y, x)

    out = pl.pallas_call(
        partial(_body, n=n, p=p, zb=zb, n_zb=n_zb, y=y, x=x),
        out_shape=jax.ShapeDtypeStruct((1, 1), jnp.float32),
        grid=(n, n_zb),
        in_specs=[
            pl.BlockSpec((n, p, 3), lambda i, j: (0, 0, 0)),
            pl.BlockSpec((1, zb, y, x), lambda i, j: (i, j, 0, 0)),
        ],
        out_specs=pl.BlockSpec((1, 1), lambda i, j: (0, 0)),
        scratch_shapes=[pltpu.VMEM((1, p), jnp.float32)],
        compiler_params=pltpu.CompilerParams(
            dimension_semantics=("arbitrary", "arbitrary"),
        ),
        name="soft_assign_point_loss",
    )(pts, logits4)

    return out[0, 0]


# speedup vs baseline: 1.2745x; 1.2745x over previous
import jax
import jax.numpy as jnp
from jax.experimental import pallas as pl
from jax.experimental.pallas import tpu as pltpu
from functools import partial

_SIGMA = 2.0
_EPS = 1e-8
_INV2S2 = 1.0 / (2.0 * _SIGMA * _SIGMA)


def _body(pts_ref, logits_ref, out_ref, *, n, p, z, y, x):
    i = pl.program_id(0)

    pts = pts_ref[i]
    pz = pts[:, 0].reshape(1, p)
    py = pts[:, 1].reshape(1, p)
    px = pts[:, 2].reshape(1, p)

    zvals = jax.lax.broadcasted_iota(jnp.int32, (z, 1), 0).astype(jnp.float32)
    kz = jnp.exp(-((zvals - pz) ** 2) * _INV2S2)
    yvals = jax.lax.broadcasted_iota(jnp.int32, (y, 1), 0).astype(jnp.float32)
    ky = jnp.exp(-((yvals - py) ** 2) * _INV2S2)
    xvals = jax.lax.broadcasted_iota(jnp.int32, (x, 1), 0).astype(jnp.float32)
    kx = jnp.exp(-((xvals - px) ** 2) * _INV2S2)

    a = (kz[:, None, :] * ky[None, :, :]).reshape(z * y, p)

    denom = jax.lax.dot_general(
        a, kx, (((1,), (1,)), ((), ())),
        preferred_element_type=jnp.float32,
    )

    l = logits_ref[0].reshape(z * y, x)
    e = jnp.exp2(l * (-1.4426950408889634))
    w = 1.0 / ((1.0 + e) * jnp.maximum(denom, _EPS))

    t = jax.lax.dot_general(
        w, kx, (((1,), (0,)), ((), ())),
        preferred_element_type=jnp.float32,
    )
    m = jnp.sum(a * t, axis=0, keepdims=True)

    pt = -jnp.log(jnp.maximum(m, _EPS))
    loss_i = jnp.sum(pt) * (1.0 / (p * n))

    @pl.when(i == 0)
    def _():
        out_ref[...] = jnp.zeros_like(out_ref)

    out_ref[...] += loss_i


@jax.jit
def kernel(logits, pts):
    n, _, z, y, x = logits.shape
    p = pts.shape[1]
    logits4 = logits.reshape(n, z, y, x)

    out = pl.pallas_call(
        partial(_body, n=n, p=p, z=z, y=y, x=x),
        out_shape=jax.ShapeDtypeStruct((1, 1), jnp.float32),
        grid=(n,),
        in_specs=[
            pl.BlockSpec((n, p, 3), lambda i: (0, 0, 0)),
            pl.BlockSpec((1, z, y, x), lambda i: (i, 0, 0, 0)),
        ],
        out_specs=pl.BlockSpec((1, 1), lambda i: (0, 0)),
        compiler_params=pltpu.CompilerParams(
            dimension_semantics=("arbitrary",),
        ),
        name="soft_assign_point_loss",
    )(pts, logits4)

    return out[0, 0]
